# Initial kernel scaffold; baseline (speedup 1.0000x reference)
import jax
import jax.numpy as jnp
from jax.experimental import pallas as pl


def kernel(node_type, velocity, mesh_pos, cells, is_training, params):
    raise NotImplementedError("write your pallas kernel here")


# speedup vs baseline: 3.2513x; 3.2513x over previous
import functools

import jax
import jax.numpy as jnp
from jax import lax
from jax.experimental import pallas as pl
from jax.experimental.pallas import tpu as pltpu
from jax.experimental.pallas import tpu_sc as plsc

N = 50000
NTS = 9
L = 32
E = 300000
NW = 32
CHUNK = 9600
P = NW * CHUNK
KB = 128
NK = CHUNK // KB
NK2 = 2 * NK
TAB = N * (N + 1) // 2
TSIZE = TAB + 8
DUMMY = N
JW = 80
JW1 = 64
ACC_ROWS = 50016
STRIPE = ACC_ROWS // 16
XSTRIPE = N // 16

def _wid():
    return lax.axis_index("s") * 2 + lax.axis_index("c")


def _lazy(builder):
    cache = []

    def call(*args):
        if not cache:
            cache.append(builder())
        return cache[0](*args)

    return call


def _build_sc_scatter_ids():
  mesh = plsc.VectorSubcoreMesh(core_axis_name="c", subcore_axis_name="s")

  @functools.partial(
      pl.kernel,
      out_type=jax.ShapeDtypeStruct((TSIZE,), jnp.int32),
      mesh=mesh,
      compiler_params=pltpu.CompilerParams(use_tc_tiling_on_sc=False),
      scratch_types=[
          pltpu.VMEM((NK, KB), jnp.int32),
          pltpu.VMEM((NK, KB), jnp.int32),
          pltpu.SemaphoreType.DMA,
      ],
  )
  def _sc_scatter_ids(t_hbm, ids_hbm, table_hbm, idx_v, val_v, sem):
    w = _wid()
    pltpu.sync_copy(t_hbm.at[w], idx_v)
    pltpu.sync_copy(ids_hbm.at[w], val_v)

    def body(j, carry):
        pltpu.async_copy(val_v.at[j], table_hbm.at[idx_v.at[j]], sem).wait()
        return carry

    lax.fori_loop(0, NK, body, 0)

  return _sc_scatter_ids


_sc_scatter_ids = _lazy(_build_sc_scatter_ids)


def _build_sc_mark_reps():
  mesh = plsc.VectorSubcoreMesh(core_axis_name="c", subcore_axis_name="s")

  @functools.partial(
      pl.kernel,
      out_type=[
          jax.ShapeDtypeStruct((NW, NK, KB), jnp.int32),
          jax.ShapeDtypeStruct((NW, NK, KB), jnp.int32),
      ],
      mesh=mesh,
      compiler_params=pltpu.CompilerParams(use_tc_tiling_on_sc=False),
      scratch_types=[
          pltpu.VMEM((NK, KB), jnp.int32),
          pltpu.VMEM((NK, KB), jnp.int32),
          pltpu.VMEM((NK, KB), jnp.int32),
          pltpu.VMEM((NK, KB), jnp.int32),
          pltpu.VMEM((NK, KB), jnp.int32),
          pltpu.VMEM((NK, KB), jnp.int32),
          pltpu.SemaphoreType.DMA,
      ],
  )
  def _sc_mark_reps(t_hbm, lo_hbm, hi_hbm, table_hbm, r1_hbm, r2_hbm,
                    idx_v, w_v, lo_v, hi_v, r1_v, r2_v, sem):
    w = _wid()
    pltpu.sync_copy(t_hbm.at[w], idx_v)
    pltpu.sync_copy(lo_hbm.at[w], lo_v)
    pltpu.sync_copy(hi_hbm.at[w], hi_v)

    def gbody(j, carry):
        pltpu.async_copy(table_hbm.at[idx_v.at[j]], w_v.at[j], sem).wait()
        return carry

    lax.fori_loop(0, NK, gbody, 0)

    base = w * CHUNK
    lanes = lax.broadcasted_iota(jnp.int32, (16,), 0)

    def cbody(i, carry):
        j = i // 8
        o = (i % 8) * 16
        wv = w_v[j, pl.ds(o, 16)]
        lov = lo_v[j, pl.ds(o, 16)]
        hiv = hi_v[j, pl.ds(o, 16)]
        pos = base + i * 16 + lanes
        m = (wv == pos) & (pos < E)
        r1_v[j, pl.ds(o, 16)] = jnp.where(m, hiv, DUMMY)
        r2_v[j, pl.ds(o, 16)] = jnp.where(m, lov, DUMMY)
        return carry

    lax.fori_loop(0, CHUNK // 16, cbody, 0)
    pltpu.sync_copy(r1_v, r1_hbm.at[w])
    pltpu.sync_copy(r2_v, r2_hbm.at[w])

  return _sc_mark_reps


_sc_mark_reps = _lazy(_build_sc_mark_reps)


def _make_sc_gather(width):
    mesh = plsc.VectorSubcoreMesh(core_axis_name="c", subcore_axis_name="s")

    @functools.partial(
        pl.kernel,
        out_type=[
            jax.ShapeDtypeStruct((P, width), jnp.float32),
            jax.ShapeDtypeStruct((P, width), jnp.float32),
        ],
        mesh=mesh,
        compiler_params=pltpu.CompilerParams(use_tc_tiling_on_sc=False),
        scratch_types=[
            pltpu.VMEM((NK, KB), jnp.int32),
            pltpu.VMEM((NK, KB), jnp.int32),
            pltpu.VMEM((KB, width), jnp.float32),
            pltpu.VMEM((KB, width), jnp.float32),
            pltpu.SemaphoreType.DMA,
            pltpu.SemaphoreType.DMA,
        ],
    )
    def _sc_gather(j_hbm, lo_hbm, hi_hbm, jlo_hbm, jhi_hbm,
                   loidx_v, hiidx_v, bufa, bufb, sema, semb):
        w = _wid()
        pltpu.sync_copy(lo_hbm.at[w], loidx_v)
        pltpu.sync_copy(hi_hbm.at[w], hiidx_v)

        def body(j, carry):
            row0 = (w * NK + j) * KB
            cpa = pltpu.async_copy(j_hbm.at[loidx_v.at[j]], bufa, sema)
            cpb = pltpu.async_copy(j_hbm.at[hiidx_v.at[j]], bufb, semb)
            cpa.wait()
            pltpu.sync_copy(bufa, jlo_hbm.at[pl.ds(row0, KB)])
            cpb.wait()
            pltpu.sync_copy(bufb, jhi_hbm.at[pl.ds(row0, KB)])
            return carry

        lax.fori_loop(0, NK, body, 0)

    return _sc_gather


_sc_gather_j0 = _lazy(lambda: _make_sc_gather(JW))
_sc_gather_j1 = _lazy(lambda: _make_sc_gather(JW1))


def _build_sc_segsum():
  mesh = plsc.VectorSubcoreMesh(core_axis_name="c", subcore_axis_name="s")

  @functools.partial(
      pl.kernel,
      out_type=jax.ShapeDtypeStruct((2, N, L), jnp.float32),
      mesh=mesh,
      compiler_params=pltpu.CompilerParams(use_tc_tiling_on_sc=False),
      scratch_types=[
          pltpu.VMEM((NK2, KB), jnp.int32),
          pltpu.VMEM((KB, L), jnp.float32),
          pltpu.VMEM_SHARED((ACC_ROWS, L), jnp.float32),
          pltpu.SemaphoreType.DMA,
      ],
  )
  def _sc_segsum(he_hbm, recv_hbm, zeros_hbm, agg_hbm, idx_v, he_v, acc, sem):
    c = lax.axis_index("c")
    s = lax.axis_index("s")
    w = s * 2 + c
    pltpu.sync_copy(recv_hbm.at[w], idx_v)
    pltpu.sync_copy(zeros_hbm.at[pl.ds(s * STRIPE, STRIPE)],
                    acc.at[pl.ds(s * STRIPE, STRIPE)])
    plsc.subcore_barrier()

    def body(j, carry):
        row0 = (w * NK2 + j) * KB
        pltpu.async_copy(he_hbm.at[pl.ds(row0, KB)], he_v, sem).wait()
        pltpu.sync_copy(he_v, acc.at[idx_v.at[j]], add=True)
        return carry

    lax.fori_loop(0, NK2, body, 0)
    plsc.subcore_barrier()
    pltpu.sync_copy(acc.at[pl.ds(s * XSTRIPE, XSTRIPE)],
                    agg_hbm.at[c].at[pl.ds(s * XSTRIPE, XSTRIPE)])

  return _sc_segsum


_sc_segsum = _lazy(_build_sc_segsum)


RN = 2000
RE = 2048


def _tc1_body(nt_ref, vel_ref, mp_ref, nmean, nstd, enW1, enb1, enW2, enb2,
              Wa, Wb, h_ref, j_ref):
    nt = nt_ref[...]
    oh = (lax.broadcasted_iota(jnp.int32, (RN, NTS), 1) == nt).astype(jnp.float32)
    nf = jnp.concatenate([vel_ref[...], oh], axis=1)
    nfn = (nf - nmean[...]) / nstd[...]
    h = jnp.dot(jax.nn.relu(jnp.dot(nfn, enW1[...]) + enb1[...]),
                enW2[...]) + enb2[...]
    h_ref[...] = h
    z = jnp.zeros((RN, JW - 66), jnp.float32)
    j_ref[...] = jnp.concatenate(
        [jnp.dot(h, Wa[...]), jnp.dot(h, Wb[...]), mp_ref[...], z], axis=1)


def _tc2_body(jlo_ref, jhi_ref, emean, estd, eeW1, eeb1, eeW2, eeb2,
              Wc, b1, W2, b2, he_ref):
    jlo = jlo_ref[...]
    jhi = jhi_ref[...]
    rel = jlo[:, 64:66] - jhi[:, 64:66]
    nrm = jnp.sqrt(jnp.sum(rel * rel, axis=1, keepdims=True))
    for d in (0, 1):
        r = rel if d == 0 else -rel
        ef = jnp.concatenate([r, nrm], axis=1)
        efn = (ef - emean[...]) / estd[...]
        he0 = jnp.dot(jax.nn.relu(jnp.dot(efn, eeW1[...]) + eeb1[...]),
                      eeW2[...]) + eeb2[...]
        if d == 0:
            pre = jlo[:, 0:32] + jhi[:, 32:64] + jnp.dot(he0, Wc[...]) + b1[...]
        else:
            pre = jhi[:, 0:32] + jlo[:, 32:64] + jnp.dot(he0, Wc[...]) + b1[...]
        he_ref[d] = he0 + jnp.dot(jax.nn.relu(pre), W2[...]) + b2[...]


def _tc3_body(h_ref, agg_ref, Wh, Wg, bn1, Wn2, bn2, Wa, Wb, h1_ref, j_ref):
    h = h_ref[...]
    agg = agg_ref[0] + agg_ref[1]
    pre = jnp.dot(h, Wh[...]) + jnp.dot(agg, Wg[...]) + bn1[...]
    h1 = h + jnp.dot(jax.nn.relu(pre), Wn2[...]) + bn2[...]
    h1_ref[...] = h1
    j_ref[...] = jnp.concatenate([jnp.dot(h1, Wa[...]), jnp.dot(h1, Wb[...])],
                                 axis=1)


def _tc4_body(jlo_ref, jhi_ref, he_ref, Wc, b1, W2, b2, heo_ref):
    jlo = jlo_ref[...]
    jhi = jhi_ref[...]
    for d in (0, 1):
        he = he_ref[d]
        if d == 0:
            pre = jlo[:, 0:32] + jhi[:, 32:64] + jnp.dot(he, Wc[...]) + b1[...]
        else:
            pre = jhi[:, 0:32] + jlo[:, 32:64] + jnp.dot(he, Wc[...]) + b1[...]
        heo_ref[d] = he + jnp.dot(jax.nn.relu(pre), W2[...]) + b2[...]


def _tc5_body(h_ref, agg_ref, vel_ref, Wh, Wg, bn1, Wn2, bn2,
              dW1, db1, dW2, db2, omean, ostd, out_ref, upd_ref):
    h = h_ref[...]
    agg = agg_ref[0] + agg_ref[1]
    pre = jnp.dot(h, Wh[...]) + jnp.dot(agg, Wg[...]) + bn1[...]
    h2 = h + jnp.dot(jax.nn.relu(pre), Wn2[...]) + bn2[...]
    o = jnp.dot(jax.nn.relu(jnp.dot(h2, dW1[...]) + db1[...]), dW2[...]) + db2[...]
    out_ref[...] = o
    upd_ref[...] = vel_ref[...] + o * ostd[...] + omean[...]


def _full(shape):
    return pl.BlockSpec(shape, lambda i: tuple(0 for _ in shape))


def _rows(shape):
    return pl.BlockSpec(shape, lambda i: (i,) + tuple(0 for _ in shape[1:]))


def _rows3(shape):
    return pl.BlockSpec(shape, lambda i: (0, i, 0))


def _tc_call(body, grid, in_specs, out_specs, out_shape):
    return pl.pallas_call(body, grid=(grid,), in_specs=in_specs,
                          out_specs=out_specs, out_shape=out_shape)


def _r2(v):
    return v.reshape(1, -1)


def kernel(node_type, velocity, mesh_pos, cells, is_training, params):
    p = params
    f32 = jnp.float32

    e = jnp.concatenate([cells[:, 0:2], cells[:, 1:3],
                         jnp.stack([cells[:, 2], cells[:, 0]], axis=1)], axis=0)
    lo = jnp.minimum(e[:, 0], e[:, 1])
    hi = jnp.maximum(e[:, 0], e[:, 1])
    tri = jnp.where(lo % 2 == 0, (lo // 2) * (lo + 1), lo * ((lo + 1) // 2))
    t = lo * N - tri + hi
    pad = P - E
    t_pad = jnp.concatenate([t, jnp.full((pad,), TAB, jnp.int32)])
    lo_pad = jnp.concatenate([lo, jnp.zeros((pad,), jnp.int32)])
    hi_pad = jnp.concatenate([hi, jnp.zeros((pad,), jnp.int32)])
    ids = jnp.arange(P, dtype=jnp.int32)

    t3 = t_pad.reshape(NW, NK, KB)
    ids3 = ids.reshape(NW, NK, KB)
    lo3 = lo_pad.reshape(NW, NK, KB)
    hi3 = hi_pad.reshape(NW, NK, KB)

    table = _sc_scatter_ids(t3, ids3)
    r1, r2 = _sc_mark_reps(t3, lo3, hi3, table)
    recv = jnp.concatenate([r1.reshape(P), r2.reshape(P)]).reshape(NW, NK2, KB)

    w0 = p['We1'][0]
    h0, j0 = _tc_call(
        _tc1_body, N // RN,
        [_rows((RN, 1)), _rows((RN, 2)), _rows((RN, 2)),
         _full((1, NTS + 2)), _full((1, NTS + 2)),
         _full((NTS + 2, L)), _full((1, L)), _full((L, L)), _full((1, L)),
         _full((L, L)), _full((L, L))],
        [_rows((RN, L)), _rows((RN, JW))],
        [jax.ShapeDtypeStruct((N, L), f32), jax.ShapeDtypeStruct((N, JW), f32)],
    )(node_type, velocity, mesh_pos, _r2(p['node_mean']), _r2(p['node_std']),
      p['enW1'], _r2(p['enb1']), p['enW2'], _r2(p['enb2']),
      w0[0:L], w0[L:2 * L])

    jlo0, jhi0 = _sc_gather_j0(j0, lo3, hi3)

    he1 = _tc_call(
        _tc2_body, P // RE,
        [_rows((RE, JW)), _rows((RE, JW)),
         _full((1, 3)), _full((1, 3)), _full((3, L)), _full((1, L)),
         _full((L, L)), _full((1, L)), _full((L, L)), _full((1, L)),
         _full((L, L)), _full((1, L))],
        _rows3((2, RE, L)),
        jax.ShapeDtypeStruct((2, P, L), f32),
    )(jlo0, jhi0, _r2(p['edge_mean']), _r2(p['edge_std']), p['eeW1'],
      _r2(p['eeb1']), p['eeW2'], _r2(p['eeb2']), w0[2 * L:3 * L],
      _r2(p['be1'][0]), p['We2'][0], _r2(p['be2'][0]))

    zeros = jnp.zeros((ACC_ROWS, L), f32)

    agg0 = _sc_segsum(he1.reshape(2 * P, L), recv, zeros)

    wn0 = p['Wn1'][0]
    w1 = p['We1'][1]
    h1, j1 = _tc_call(
        _tc3_body, N // RN,
        [_rows((RN, L)), _rows3((2, RN, L)),
         _full((L, L)), _full((L, L)), _full((1, L)), _full((L, L)), _full((1, L)),
         _full((L, L)), _full((L, L))],
        [_rows((RN, L)), _rows((RN, JW1))],
        [jax.ShapeDtypeStruct((N, L), f32), jax.ShapeDtypeStruct((N, JW1), f32)],
    )(h0, agg0, wn0[0:L], wn0[L:2 * L], _r2(p['bn1'][0]), p['Wn2'][0],
      _r2(p['bn2'][0]), w1[0:L], w1[L:2 * L])

    jlo1, jhi1 = _sc_gather_j1(j1, lo3, hi3)

    he2 = _tc_call(
        _tc4_body, P // RE,
        [_rows((RE, JW1)), _rows((RE, JW1)), _rows3((2, RE, L)),
         _full((L, L)), _full((1, L)), _full((L, L)), _full((1, L))],
        _rows3((2, RE, L)),
        jax.ShapeDtypeStruct((2, P, L), f32),
    )(jlo1, jhi1, he1, w1[2 * L:3 * L], _r2(p['be1'][1]), p['We2'][1],
      _r2(p['be2'][1]))

    agg1 = _sc_segsum(he2.reshape(2 * P, L), recv, zeros)

    wn1 = p['Wn1'][1]
    out, updated = _tc_call(
        _tc5_body, N // RN,
        [_rows((RN, L)), _rows3((2, RN, L)), _rows((RN, 2)),
         _full((L, L)), _full((L, L)), _full((1, L)), _full((L, L)), _full((1, L)),
         _full((L, L)), _full((1, L)), _full((L, 2)), _full((1, 2)),
         _full((1, 2)), _full((1, 2))],
        [_rows((RN, 2)), _rows((RN, 2))],
        [jax.ShapeDtypeStruct((N, 2), f32), jax.ShapeDtypeStruct((N, 2), f32)],
    )(h1, agg1, velocity, wn1[0:L], wn1[L:2 * L], _r2(p['bn1'][1]),
      p['Wn2'][1], _r2(p['bn2'][1]), p['dW1'], _r2(p['db1']), p['dW2'],
      _r2(p['db2']), _r2(p['out_mean']), _r2(p['out_std']))

    return jnp.where(is_training != 0, out, updated)
